# baseline (speedup 1.0000x reference)
import functools

import jax
import jax.numpy as jnp
from jax import lax
from jax.experimental import pallas as pl
from jax.experimental.pallas import tpu as pltpu
from jax.experimental.pallas import tpu_sc as plsc

N = 10000
E = 320000
D = 128
G = 64

NC = 2
NS = 16
L = 16

EA = E // NS
EB = E // (NC * NS)
CHUNK = 80
NCH = EB // CHUNK
NPAD = 10240
DEN_R = NPAD // D

_f32 = jnp.float32
_i32 = jnp.int32


def _tc_in_layer(x, W, apad):
    def body(x_ref, w_ref, a_ref, h_ref, s_ref):
        h = jnp.dot(x_ref[...], w_ref[...], preferred_element_type=_f32)
        h_ref[...] = h
        s_ref[...] = jnp.dot(h, a_ref[...], preferred_element_type=_f32)

    return pl.pallas_call(
        body,
        grid=(10,),
        in_specs=[
            pl.BlockSpec((1000, D), lambda i: (i, 0)),
            pl.BlockSpec((D, D), lambda i: (0, 0)),
            pl.BlockSpec((D, D), lambda i: (0, 0)),
        ],
        out_specs=[
            pl.BlockSpec((1000, D), lambda i: (i, 0)),
            pl.BlockSpec((1000, D), lambda i: (i, 0)),
        ],
        out_shape=[
            jax.ShapeDtypeStruct((N, D), _f32),
            jax.ShapeDtypeStruct((N, D), _f32),
        ],
    )(x, W, apad)


def _tc_mid_layer(p0, p1, b2d, W, apad):
    def body(p0_ref, p1_ref, b_ref, w_ref, a_ref, h_ref, s_ref):
        hin = jax.nn.relu(p0_ref[...] + p1_ref[...] + b_ref[...])
        h = jnp.dot(hin, w_ref[...], preferred_element_type=_f32)
        h_ref[...] = h
        s_ref[...] = jnp.dot(h, a_ref[...], preferred_element_type=_f32)

    return pl.pallas_call(
        body,
        grid=(10,),
        in_specs=[
            pl.BlockSpec((1000, D), lambda i: (i, 0)),
            pl.BlockSpec((1000, D), lambda i: (i, 0)),
            pl.BlockSpec((1, D), lambda i: (0, 0)),
            pl.BlockSpec((D, D), lambda i: (0, 0)),
            pl.BlockSpec((D, D), lambda i: (0, 0)),
        ],
        out_specs=[
            pl.BlockSpec((1000, D), lambda i: (i, 0)),
            pl.BlockSpec((1000, D), lambda i: (i, 0)),
        ],
        out_shape=[
            jax.ShapeDtypeStruct((N, D), _f32),
            jax.ShapeDtypeStruct((N, D), _f32),
        ],
    )(p0, p1, b2d, W, apad)


def _tc_pool(p0, p1, b2d, batch_pad):
    NP = batch_pad.shape[1]

    def body(p0_ref, p1_ref, b_ref, bat_ref, o_ref):
        h = p0_ref[...] + p1_ref[...] + b_ref[...]
        gid = lax.broadcasted_iota(_i32, (G, NP), 0)
        m = (bat_ref[...] == gid).astype(_f32)
        sums = jnp.dot(m, h, preferred_element_type=_f32)
        counts = jnp.sum(m, axis=1)
        pooled = sums / jnp.maximum(counts, 1.0)[:, None]
        o_ref[...] = jax.nn.sigmoid(pooled)

    return pl.pallas_call(
        body,
        grid=(1,),
        in_specs=[
            pl.BlockSpec((NP, D), lambda i: (0, 0)),
            pl.BlockSpec((NP, D), lambda i: (0, 0)),
            pl.BlockSpec((1, D), lambda i: (0, 0)),
            pl.BlockSpec((1, NP), lambda i: (0, 0)),
        ],
        out_specs=pl.BlockSpec((G, D), lambda i: (0, 0)),
        out_shape=jax.ShapeDtypeStruct((G, D), _f32),
    )(p0, p1, b2d, batch_pad)


def _sc_edge_softmax(src1d, dst1d, zblk, asrc, adst):
    mesh = plsc.VectorSubcoreMesh(core_axis_name="c", subcore_axis_name="s")

    @functools.partial(
        pl.kernel,
        out_type=jax.ShapeDtypeStruct((E,), _f32),
        mesh=mesh,
        scratch_types=dict(
            as_v=pltpu.VMEM((N,), _f32),
            ad_v=pltpu.VMEM((N,), _f32),
            srcA_v=pltpu.VMEM((EA,), _i32),
            dstA_v=pltpu.VMEM((EA,), _i32),
            eeA_v=pltpu.VMEM((EA,), _f32),
            den_v=pltpu.VMEM((DEN_R, D), _f32),
            rix_v=pltpu.VMEM((1, DEN_R), _i32),
            den_s=pltpu.VMEM_SHARED((DEN_R, D), _f32),
        ),
        compiler_params=pltpu.CompilerParams(needs_layout_passes=False),
    )
    def k(src_hbm, dst_hbm, z_hbm, asrc_hbm, adst_hbm, out_hbm, *,
          as_v, ad_v, srcA_v, dstA_v, eeA_v, den_v, rix_v, den_s):
        c0 = lax.axis_index("c")
        s0 = lax.axis_index("s")

        pltpu.sync_copy(z_hbm.at[pl.ds(0, DEN_R)], den_v)

        @pl.when(s0 == 0)
        def _():
            pltpu.sync_copy(den_v, den_s)

        pltpu.sync_copy(asrc_hbm, as_v)
        pltpu.sync_copy(adst_hbm, ad_v)
        pltpu.sync_copy(src_hbm.at[pl.ds(EA * s0, EA)], srcA_v)
        pltpu.sync_copy(dst_hbm.at[pl.ds(EA * s0, EA)], dstA_v)

        for j in range(DEN_R // L):
            rix_v[0, pl.ds(L * j, L)] = lax.iota(_i32, L) + (L * j)

        plsc.subcore_barrier()

        def edge_body(i, _):
            sv = srcA_v[pl.ds(i * L, L)]
            dv = dstA_v[pl.ds(i * L, L)]
            e = plsc.load_gather(as_v, [sv]) + plsc.load_gather(ad_v, [dv])
            e = jnp.where(e > 0, e, 0.2 * e)
            ee = jnp.exp(e)
            eeA_v[pl.ds(i * L, L)] = ee
            plsc.addupdate_scatter(
                den_v,
                [lax.shift_right_logical(dv, 7), jnp.bitwise_and(dv, 127)],
                ee,
            )
            return 0

        lax.fori_loop(0, EA // L, edge_body, 0)

        pltpu.sync_copy(den_v, den_s.at[rix_v.at[0]], add=True)
        plsc.subcore_barrier()
        pltpu.sync_copy(den_s, den_v)

        boff = EB * c0

        def alpha_body(i, _):
            off = boff + i * L
            dv = dstA_v[pl.ds(off, L)]
            dnm = plsc.load_gather(
                den_v,
                [lax.shift_right_logical(dv, 7), jnp.bitwise_and(dv, 127)],
            )
            eeA_v[pl.ds(off, L)] = eeA_v[pl.ds(off, L)] / (dnm + 1e-16)
            return 0

        lax.fori_loop(0, EB // L, alpha_body, 0)

        pltpu.sync_copy(
            eeA_v.at[pl.ds(boff, EB)],
            out_hbm.at[pl.ds(EA * s0 + boff, EB)],
        )

    return k(src1d, dst1d, zblk, asrc, adst)


BS = 2000
NBLK = EB // BS
NCHB = BS // CHUNK


def _sc_row_accum(src1d, dst1d, zblk, alpha, h):
    mesh = plsc.VectorSubcoreMesh(core_axis_name="c", subcore_axis_name="s")

    @functools.partial(
        pl.kernel,
        out_type=jax.ShapeDtypeStruct((NC * NPAD, D), _f32),
        mesh=mesh,
        scratch_types=dict(
            bsrc=pltpu.VMEM((BS,), _i32),
            bdst=pltpu.VMEM((BS,), _i32),
            balp=pltpu.VMEM((BS,), _f32),
            dstB_v=pltpu.VMEM((3, CHUNK), _i32),
            rows0=pltpu.VMEM((CHUNK, D), _f32),
            rows1=pltpu.VMEM((CHUNK, D), _f32),
            rows2=pltpu.VMEM((CHUNK, D), _f32),
            acc_s=pltpu.VMEM_SHARED((NPAD, D), _f32),
            gsem0=pltpu.SemaphoreType.DMA,
            gsem1=pltpu.SemaphoreType.DMA,
            gsem2=pltpu.SemaphoreType.DMA,
            ssem0=pltpu.SemaphoreType.DMA,
            ssem1=pltpu.SemaphoreType.DMA,
            ssem2=pltpu.SemaphoreType.DMA,
        ),
        compiler_params=pltpu.CompilerParams(needs_layout_passes=False),
    )
    def k(src_hbm, dst_hbm, z_hbm, alpha_hbm, h_hbm, out_hbm, *,
          bsrc, bdst, balp, dstB_v, rows0, rows1, rows2, acc_s,
          gsem0, gsem1, gsem2, ssem0, ssem1, ssem2):
        c0 = lax.axis_index("c")
        s0 = lax.axis_index("s")
        ebase = EA * s0 + EB * c0

        for z in range(NPAD // NS // D):
            pltpu.sync_copy(z_hbm, acc_s.at[pl.ds(640 * s0 + D * z, D)])
        plsc.subcore_barrier()

        rows_b = (rows0, rows1, rows2)
        gsem_b = (gsem0, gsem1, gsem2)
        ssem_b = (ssem0, ssem1, ssem2)

        def g_start(cc, par):
            pltpu.async_copy(
                h_hbm.at[bsrc.at[pl.ds(CHUNK * cc, CHUNK)]],
                rows_b[par], gsem_b[par])

        def g_wait(cc, par):
            pltpu.make_async_copy(
                h_hbm.at[bsrc.at[pl.ds(CHUNK * cc, CHUNK)]],
                rows_b[par], gsem_b[par]).wait()

        def s_start(par):
            pltpu.async_copy(
                rows_b[par], acc_s.at[dstB_v.at[par]], ssem_b[par], add=True)

        def s_wait(par):
            pltpu.make_async_copy(
                rows_b[par], acc_s.at[dstB_v.at[par]], ssem_b[par]).wait()

        def scale(cc, par):
            rows = rows_b[par]
            cbase = CHUNK * cc
            for j in range(CHUNK // L):
                dstB_v[par, pl.ds(L * j, L)] = bdst[pl.ds(cbase + L * j, L)]

            def scale16(g2, _):
                base = cbase + L * g2
                av = balp[pl.ds(base, L)]
                for e in range(0, L, 2):
                    sp0 = av.at[jnp.full((L,), e, _i32)].get(
                        mode="promise_in_bounds")
                    sp1 = av.at[jnp.full((L,), e + 1, _i32)].get(
                        mode="promise_in_bounds")
                    ea = L * g2 + e
                    eb = ea + 1
                    for q in range(D // L):
                        ra = rows[ea, pl.ds(L * q, L)]
                        rb = rows[eb, pl.ds(L * q, L)]
                        rows[ea, pl.ds(L * q, L)] = ra * sp0
                        rows[eb, pl.ds(L * q, L)] = rb * sp1
                return 0

            lax.fori_loop(0, CHUNK // L, scale16, 0)

        NT = NCHB // 3
        for blk in range(NBLK):
            bo = ebase + BS * blk
            pltpu.sync_copy(src_hbm.at[pl.ds(bo, BS)], bsrc)
            pltpu.sync_copy(dst_hbm.at[pl.ds(bo, BS)], bdst)
            pltpu.sync_copy(alpha_hbm.at[pl.ds(bo, BS)], balp)

            g_start(0, 0)
            g_start(1, 1)

            def triple_body(t, _):
                for kk in range(3):
                    cc = 3 * t + kk
                    g_wait(cc, kk)
                    scale(cc, kk)
                    nxt = (kk + 2) % 3
                    if kk == 0:
                        @pl.when(t > 0)
                        def _():
                            s_wait(nxt)
                        g_start(cc + 2, nxt)
                    elif kk == 1:
                        s_wait(nxt)
                        g_start(cc + 2, nxt)
                    else:
                        s_wait(nxt)

                        @pl.when(t < NT - 1)
                        def _():
                            g_start(cc + 2, nxt)
                    s_start(kk)
                return 0

            lax.fori_loop(0, NT, triple_body, 0)

            g_wait(NCHB - 1, 0)
            scale(NCHB - 1, 0)
            s_start(0)

            s_wait(2)
            s_wait(0)

        plsc.subcore_barrier()

        for w in range(NPAD // NS // D):
            pltpu.sync_copy(
                acc_s.at[pl.ds(640 * s0 + D * w, D)],
                out_hbm.at[pl.ds(NPAD * c0 + 640 * s0 + D * w, D)],
            )

    return k(src1d, dst1d, zblk, alpha, h)


def kernel(x, edge_index, batch, W1, a1_src, a1_dst, b1, W2, a2_src, a2_dst, b2):
    src = edge_index[0]
    dst = edge_index[1]
    zblk = jnp.zeros((D, D), _f32)

    def apack(a_s, a_d):
        ap = jnp.zeros((D, D), _f32)
        return ap.at[:, 0].set(a_s).at[:, 1].set(a_d)

    h1, s1 = _tc_in_layer(x, W1, apack(a1_src, a1_dst))
    al1 = _sc_edge_softmax(src, dst, zblk, s1[:, 0], s1[:, 1])
    acc1 = _sc_row_accum(src, dst, zblk, al1, h1)
    h2, s2 = _tc_mid_layer(acc1[:N], acc1[NPAD:NPAD + N], b1.reshape(1, D),
                           W2, apack(a2_src, a2_dst))
    al2 = _sc_edge_softmax(src, dst, zblk, s2[:, 0], s2[:, 1])
    acc2 = _sc_row_accum(src, dst, zblk, al2, h2)

    p0 = jnp.pad(acc2[:N], ((0, NPAD - N), (0, 0)))
    p1 = jnp.pad(acc2[NPAD:NPAD + N], ((0, NPAD - N), (0, 0)))
    batch_pad = jnp.pad(batch, (0, NPAD - N), constant_values=G).reshape(1, NPAD)
    return _tc_pool(p0, p1, b2.reshape(1, D), batch_pad)


# speedup vs baseline: 1.0195x; 1.0195x over previous
import functools

import jax
import jax.numpy as jnp
from jax import lax
from jax.experimental import pallas as pl
from jax.experimental.pallas import tpu as pltpu
from jax.experimental.pallas import tpu_sc as plsc

N = 10000
E = 320000
D = 128
G = 64

NC = 2
NS = 16
L = 16

EA = E // NS
EB = E // (NC * NS)
CHUNK = 80
NCH = EB // CHUNK
NPAD = 10240
DEN_R = NPAD // D

_f32 = jnp.float32
_i32 = jnp.int32


def _tc_in_layer(x, W, apad):
    def body(x_ref, w_ref, a_ref, h_ref, s_ref):
        h = jnp.dot(x_ref[...], w_ref[...], preferred_element_type=_f32)
        h_ref[...] = h
        s_ref[...] = jnp.dot(h, a_ref[...], preferred_element_type=_f32)

    return pl.pallas_call(
        body,
        grid=(10,),
        in_specs=[
            pl.BlockSpec((1000, D), lambda i: (i, 0)),
            pl.BlockSpec((D, D), lambda i: (0, 0)),
            pl.BlockSpec((D, D), lambda i: (0, 0)),
        ],
        out_specs=[
            pl.BlockSpec((1000, D), lambda i: (i, 0)),
            pl.BlockSpec((1000, D), lambda i: (i, 0)),
        ],
        out_shape=[
            jax.ShapeDtypeStruct((N, D), _f32),
            jax.ShapeDtypeStruct((N, D), _f32),
        ],
    )(x, W, apad)


def _tc_mid_layer(p0, p1, b2d, W, apad):
    def body(p0_ref, p1_ref, b_ref, w_ref, a_ref, h_ref, s_ref):
        hin = jax.nn.relu(p0_ref[...] + p1_ref[...] + b_ref[...])
        h = jnp.dot(hin, w_ref[...], preferred_element_type=_f32)
        h_ref[...] = h
        s_ref[...] = jnp.dot(h, a_ref[...], preferred_element_type=_f32)

    return pl.pallas_call(
        body,
        grid=(10,),
        in_specs=[
            pl.BlockSpec((1000, D), lambda i: (i, 0)),
            pl.BlockSpec((1000, D), lambda i: (i, 0)),
            pl.BlockSpec((1, D), lambda i: (0, 0)),
            pl.BlockSpec((D, D), lambda i: (0, 0)),
            pl.BlockSpec((D, D), lambda i: (0, 0)),
        ],
        out_specs=[
            pl.BlockSpec((1000, D), lambda i: (i, 0)),
            pl.BlockSpec((1000, D), lambda i: (i, 0)),
        ],
        out_shape=[
            jax.ShapeDtypeStruct((N, D), _f32),
            jax.ShapeDtypeStruct((N, D), _f32),
        ],
    )(p0, p1, b2d, W, apad)


def _tc_pool(p0, p1, b2d, batch_pad):
    NP = batch_pad.shape[1]

    def body(p0_ref, p1_ref, b_ref, bat_ref, o_ref):
        h = p0_ref[...] + p1_ref[...] + b_ref[...]
        gid = lax.broadcasted_iota(_i32, (G, NP), 0)
        m = (bat_ref[...] == gid).astype(_f32)
        sums = jnp.dot(m, h, preferred_element_type=_f32)
        counts = jnp.sum(m, axis=1)
        pooled = sums / jnp.maximum(counts, 1.0)[:, None]
        o_ref[...] = jax.nn.sigmoid(pooled)

    return pl.pallas_call(
        body,
        grid=(1,),
        in_specs=[
            pl.BlockSpec((NP, D), lambda i: (0, 0)),
            pl.BlockSpec((NP, D), lambda i: (0, 0)),
            pl.BlockSpec((1, D), lambda i: (0, 0)),
            pl.BlockSpec((1, NP), lambda i: (0, 0)),
        ],
        out_specs=pl.BlockSpec((G, D), lambda i: (0, 0)),
        out_shape=jax.ShapeDtypeStruct((G, D), _f32),
    )(p0, p1, b2d, batch_pad)


def _sc_edge_softmax(src1d, dst1d, zblk, asrc, adst):
    mesh = plsc.VectorSubcoreMesh(core_axis_name="c", subcore_axis_name="s")

    @functools.partial(
        pl.kernel,
        out_type=jax.ShapeDtypeStruct((E,), _f32),
        mesh=mesh,
        scratch_types=dict(
            as_v=pltpu.VMEM((N,), _f32),
            ad_v=pltpu.VMEM((N,), _f32),
            srcA_v=pltpu.VMEM((EA,), _i32),
            dstA_v=pltpu.VMEM((EA,), _i32),
            eeA_v=pltpu.VMEM((EA,), _f32),
            den_v=pltpu.VMEM((DEN_R, D), _f32),
            rix_v=pltpu.VMEM((1, DEN_R), _i32),
            den_s=pltpu.VMEM_SHARED((DEN_R, D), _f32),
        ),
        compiler_params=pltpu.CompilerParams(needs_layout_passes=False),
    )
    def k(src_hbm, dst_hbm, z_hbm, asrc_hbm, adst_hbm, out_hbm, *,
          as_v, ad_v, srcA_v, dstA_v, eeA_v, den_v, rix_v, den_s):
        c0 = lax.axis_index("c")
        s0 = lax.axis_index("s")

        pltpu.sync_copy(z_hbm.at[pl.ds(0, DEN_R)], den_v)

        @pl.when(s0 == 0)
        def _():
            pltpu.sync_copy(den_v, den_s)

        pltpu.sync_copy(asrc_hbm, as_v)
        pltpu.sync_copy(adst_hbm, ad_v)
        pltpu.sync_copy(src_hbm.at[pl.ds(EA * s0, EA)], srcA_v)
        pltpu.sync_copy(dst_hbm.at[pl.ds(EA * s0, EA)], dstA_v)

        for j in range(DEN_R // L):
            rix_v[0, pl.ds(L * j, L)] = lax.iota(_i32, L) + (L * j)

        plsc.subcore_barrier()

        def edge_body(i, _):
            sv = srcA_v[pl.ds(i * L, L)]
            dv = dstA_v[pl.ds(i * L, L)]
            e = plsc.load_gather(as_v, [sv]) + plsc.load_gather(ad_v, [dv])
            e = jnp.where(e > 0, e, 0.2 * e)
            ee = jnp.exp(e)
            eeA_v[pl.ds(i * L, L)] = ee
            plsc.addupdate_scatter(
                den_v,
                [lax.shift_right_logical(dv, 7), jnp.bitwise_and(dv, 127)],
                ee,
            )
            return 0

        lax.fori_loop(0, EA // L, edge_body, 0)

        pltpu.sync_copy(den_v, den_s.at[rix_v.at[0]], add=True)
        plsc.subcore_barrier()
        pltpu.sync_copy(den_s, den_v)

        boff = EB * c0

        def alpha_body(i, _):
            off = boff + i * L
            dv = dstA_v[pl.ds(off, L)]
            dnm = plsc.load_gather(
                den_v,
                [lax.shift_right_logical(dv, 7), jnp.bitwise_and(dv, 127)],
            )
            eeA_v[pl.ds(off, L)] = eeA_v[pl.ds(off, L)] / (dnm + 1e-16)
            return 0

        lax.fori_loop(0, EB // L, alpha_body, 0)

        pltpu.sync_copy(
            eeA_v.at[pl.ds(boff, EB)],
            out_hbm.at[pl.ds(EA * s0 + boff, EB)],
        )

    return k(src1d, dst1d, zblk, asrc, adst)


BS = 2000
NBLK = EB // BS
NCHB = BS // CHUNK


def _sc_row_accum(src1d, dst1d, zblk, alpha, h):
    mesh = plsc.VectorSubcoreMesh(core_axis_name="c", subcore_axis_name="s")

    @functools.partial(
        pl.kernel,
        out_type=jax.ShapeDtypeStruct((NC * NPAD, D), _f32),
        mesh=mesh,
        scratch_types=dict(
            bsrc=pltpu.VMEM((BS,), _i32),
            bdst=pltpu.VMEM((BS,), _i32),
            balp=pltpu.VMEM((BS,), _f32),
            dstB_v=pltpu.VMEM((3, CHUNK), _i32),
            rows0=pltpu.VMEM((CHUNK, D), _f32),
            rows1=pltpu.VMEM((CHUNK, D), _f32),
            rows2=pltpu.VMEM((CHUNK, D), _f32),
            acc_s=pltpu.VMEM_SHARED((NPAD, D), _f32),
            gsem0=pltpu.SemaphoreType.DMA,
            gsem1=pltpu.SemaphoreType.DMA,
            gsem2=pltpu.SemaphoreType.DMA,
            ssem0=pltpu.SemaphoreType.DMA,
            ssem1=pltpu.SemaphoreType.DMA,
            ssem2=pltpu.SemaphoreType.DMA,
        ),
        compiler_params=pltpu.CompilerParams(needs_layout_passes=False),
    )
    def k(src_hbm, dst_hbm, z_hbm, alpha_hbm, h_hbm, out_hbm, *,
          bsrc, bdst, balp, dstB_v, rows0, rows1, rows2, acc_s,
          gsem0, gsem1, gsem2, ssem0, ssem1, ssem2):
        c0 = lax.axis_index("c")
        s0 = lax.axis_index("s")
        ebase = EA * s0 + EB * c0

        for z in range(NPAD // NS // D):
            pltpu.sync_copy(z_hbm, acc_s.at[pl.ds(640 * s0 + D * z, D)])
        plsc.subcore_barrier()

        rows_b = (rows0, rows1, rows2)
        gsem_b = (gsem0, gsem1, gsem2)
        ssem_b = (ssem0, ssem1, ssem2)

        def g_start(cc, par):
            pltpu.async_copy(
                h_hbm.at[bsrc.at[pl.ds(CHUNK * cc, CHUNK)]],
                rows_b[par], gsem_b[par])

        def g_wait(cc, par):
            pltpu.make_async_copy(
                h_hbm.at[bsrc.at[pl.ds(CHUNK * cc, CHUNK)]],
                rows_b[par], gsem_b[par]).wait()

        def s_start(par):
            pltpu.async_copy(
                rows_b[par], acc_s.at[dstB_v.at[par]], ssem_b[par], add=True)

        def s_wait(par):
            pltpu.make_async_copy(
                rows_b[par], acc_s.at[dstB_v.at[par]], ssem_b[par]).wait()

        def scale(cc, par):
            rows = rows_b[par]
            cbase = CHUNK * cc
            for j in range(CHUNK // L):
                dstB_v[par, pl.ds(L * j, L)] = bdst[pl.ds(cbase + L * j, L)]

            def scale16(g2, _):
                base = cbase + L * g2
                av = balp[pl.ds(base, L)]
                for e in range(L):
                    sp = av.at[jnp.full((L,), e, _i32)].get(
                        mode="promise_in_bounds")
                    e2 = L * g2 + e
                    for q in range(D // L):
                        rows[e2, pl.ds(L * q, L)] = (
                            rows[e2, pl.ds(L * q, L)] * sp)
                return 0

            lax.fori_loop(0, CHUNK // L, scale16, 0)

        NT = NCHB // 3
        for blk in range(NBLK):
            bo = ebase + BS * blk
            pltpu.sync_copy(src_hbm.at[pl.ds(bo, BS)], bsrc)
            pltpu.sync_copy(dst_hbm.at[pl.ds(bo, BS)], bdst)
            pltpu.sync_copy(alpha_hbm.at[pl.ds(bo, BS)], balp)

            g_start(0, 0)
            g_start(1, 1)

            def triple_body(t, _):
                for kk in range(3):
                    cc = 3 * t + kk
                    g_wait(cc, kk)
                    scale(cc, kk)
                    nxt = (kk + 2) % 3
                    if kk == 0:
                        @pl.when(t > 0)
                        def _():
                            s_wait(nxt)
                        g_start(cc + 2, nxt)
                    elif kk == 1:
                        s_wait(nxt)
                        g_start(cc + 2, nxt)
                    else:
                        s_wait(nxt)

                        @pl.when(t < NT - 1)
                        def _():
                            g_start(cc + 2, nxt)
                    s_start(kk)
                return 0

            lax.fori_loop(0, NT, triple_body, 0)

            g_wait(NCHB - 1, 0)
            scale(NCHB - 1, 0)
            s_start(0)

            s_wait(2)
            s_wait(0)

        plsc.subcore_barrier()

        for w in range(NPAD // NS // D):
            pltpu.sync_copy(
                acc_s.at[pl.ds(640 * s0 + D * w, D)],
                out_hbm.at[pl.ds(NPAD * c0 + 640 * s0 + D * w, D)],
            )

    return k(src1d, dst1d, zblk, alpha, h)


def kernel(x, edge_index, batch, W1, a1_src, a1_dst, b1, W2, a2_src, a2_dst, b2):
    src = edge_index[0]
    dst = edge_index[1]
    zblk = jnp.zeros((D, D), _f32)

    def apack(a_s, a_d):
        ap = jnp.zeros((D, D), _f32)
        return ap.at[:, 0].set(a_s).at[:, 1].set(a_d)

    h1, s1 = _tc_in_layer(x, W1, apack(a1_src, a1_dst))
    al1 = _sc_edge_softmax(src, dst, zblk, s1[:, 0], s1[:, 1])
    acc1 = _sc_row_accum(src, dst, zblk, al1, h1)
    h2, s2 = _tc_mid_layer(acc1[:N], acc1[NPAD:NPAD + N], b1.reshape(1, D),
                           W2, apack(a2_src, a2_dst))
    al2 = _sc_edge_softmax(src, dst, zblk, s2[:, 0], s2[:, 1])
    acc2 = _sc_row_accum(src, dst, zblk, al2, h2)

    p0 = jnp.pad(acc2[:N], ((0, NPAD - N), (0, 0)))
    p1 = jnp.pad(acc2[NPAD:NPAD + N], ((0, NPAD - N), (0, 0)))
    batch_pad = jnp.pad(batch, (0, NPAD - N), constant_values=G).reshape(1, NPAD)
    return _tc_pool(p0, p1, b2.reshape(1, D), batch_pad)
